# Initial kernel scaffold; baseline (speedup 1.0000x reference)
import jax
import jax.numpy as jnp
from jax.experimental import pallas as pl


def kernel(rv_feat, ref_bev):
    raise NotImplementedError("write your pallas kernel here")


# speedup vs baseline: 10.3675x; 10.3675x over previous
import functools

import numpy as np
import jax
import jax.numpy as jnp
from jax import lax
from jax.experimental import pallas as pl
from jax.experimental.pallas import tpu as pltpu
from jax.experimental.pallas import tpu_sc as plsc

_Hr, _Wr = 64, 2048
_Hb, _Wb = 512, 512
_R_MAX = 50.0
_VERT_ROW = 32
_B, _C = 2, 64
_N = _Hb * _Wb

_NC, _NS, _L = 2, 16, 16
_NW = _NC * _NS
_ROWS = _B * _C
_RG = 4
_PG = _NW // _RG
_ROWS_PER_W = _ROWS // _RG
_PX_PER_W = _N // _PG
_CHUNK = 8192
_NCHUNK = _PX_PER_W // _CHUNK


def _precompute_grid():
    yy, xx = np.meshgrid(np.arange(_Hb, dtype=np.float64),
                         np.arange(_Wb, dtype=np.float64), indexing="ij")
    y = (yy - _Hb / 2 + 0.5) * _R_MAX / (_Hb / 2 - 0.5)
    x = (xx - _Wb / 2 + 0.5) * _R_MAX / (_Wb / 2 - 0.5)
    phi = (np.arctan2(y, x) + 2 * np.pi) % (2 * np.pi)
    col = _Wr - 1 - phi / (2 * np.pi) * (_Wr - 1)
    ix = col * (_Wr - 1) / _Wr
    ix0 = np.floor(ix)
    w1 = (ix - ix0).astype(np.float32).reshape(-1)
    c0 = np.clip(ix0, 0, _Wr - 2).astype(np.int32).reshape(-1)
    return c0, w1


_C0_HOST, _W1_HOST = _precompute_grid()


@functools.cache
def _build_sc_kernel():
    mesh = plsc.VectorSubcoreMesh(core_axis_name="c", subcore_axis_name="s",
                                  num_cores=_NC, num_subcores=_NS)

    @functools.partial(
        pl.kernel,
        out_type=jax.ShapeDtypeStruct((_ROWS, _N), jnp.float32),
        mesh=mesh,
        compiler_params=pltpu.CompilerParams(needs_layout_passes=False),
        scratch_types=[
            pltpu.VMEM((_ROWS_PER_W * _Wr,), jnp.float32),
            pltpu.VMEM((2, _Wr), jnp.float32),
            pltpu.VMEM((_CHUNK,), jnp.int32),
            pltpu.VMEM((_CHUNK,), jnp.float32),
            pltpu.VMEM((_CHUNK,), jnp.float32),
        ],
    )
    def _rv2bev_sc(rv_hbm, c0_hbm, w_hbm, out_hbm,
                   tab_v, pair_v, idx_v, w_v, out_v):
        _sc_body(rv_hbm, c0_hbm, w_hbm, out_hbm,
                 tab_v, pair_v, idx_v, w_v, out_v)

    return _rv2bev_sc


def _sc_body(rv_hbm, c0_hbm, w_hbm, out_hbm, tab_v, pair_v, idx_v, w_v, out_v):
    wid = lax.axis_index("s") * _NC + lax.axis_index("c")
    rg = wid % _RG
    pg = wid // _RG
    row0 = rg * _ROWS_PER_W
    px0 = pg * _PX_PER_W

    def load_row(r, carry):
        pltpu.sync_copy(rv_hbm.at[row0 + r], pair_v)

        def avg_vec(i, c):
            sl = pl.ds(i * _L, _L)
            tab_v[pl.ds(r * _Wr + i * _L, _L)] = (
                0.5 * (pair_v[0, sl] + pair_v[1, sl]))
            return c

        lax.fori_loop(0, _Wr // _L, avg_vec, 0)
        return carry

    lax.fori_loop(0, _ROWS_PER_W, load_row, 0)

    def chunk_body(k, carry):
        off = px0 + k * _CHUNK
        pltpu.sync_copy(c0_hbm.at[pl.ds(off, _CHUNK)], idx_v)
        pltpu.sync_copy(w_hbm.at[pl.ds(off, _CHUNK)], w_v)

        def row_body(r, c):
            base = r * _Wr

            def vec_body(i, cc):
                sl = pl.ds(i * _L, _L)
                idx = idx_v[sl] + base
                w1 = w_v[sl]
                g0 = plsc.load_gather(tab_v, [idx])
                g1 = plsc.load_gather(tab_v, [idx + 1])
                out_v[sl] = (1.0 - w1) * g0 + w1 * g1
                return cc

            lax.fori_loop(0, _CHUNK // _L, vec_body, 0)
            pltpu.sync_copy(out_v, out_hbm.at[row0 + r, pl.ds(off, _CHUNK)])
            return c

        lax.fori_loop(0, _ROWS_PER_W, row_body, 0)
        return carry

    lax.fori_loop(0, _NCHUNK, chunk_body, 0)


def kernel(rv_feat, ref_bev):
    del ref_bev
    rv_rows = rv_feat[:, :, _VERT_ROW - 1:_VERT_ROW + 1, :].reshape(_ROWS, 2, _Wr)
    c0 = jnp.asarray(_C0_HOST)
    w1 = jnp.asarray(_W1_HOST)
    out = _build_sc_kernel()(rv_rows, c0, w1)
    return out.reshape(_B, _C, _Hb, _Wb)


# speedup vs baseline: 20.3592x; 1.9637x over previous
import functools

import numpy as np
import jax
import jax.numpy as jnp
from jax import lax
from jax.experimental import pallas as pl
from jax.experimental.pallas import tpu as pltpu
from jax.experimental.pallas import tpu_sc as plsc

_Hr, _Wr = 64, 2048
_Hb, _Wb = 512, 512
_R_MAX = 50.0
_VERT_ROW = 32
_B, _C = 2, 64
_N = _Hb * _Wb

_NC, _NS, _L = 2, 16, 16
_NW = _NC * _NS
_ROWS = _B * _C
_RG = 8
_PG = _NW // _RG
_RPW = _ROWS // _RG
_PX_PER_W = _N // _PG
_CH = 2048
_NCH = _PX_PER_W // _CH


def _precompute_grid():
    yy, xx = np.meshgrid(np.arange(_Hb, dtype=np.float64),
                         np.arange(_Wb, dtype=np.float64), indexing="ij")
    y = (yy - _Hb / 2 + 0.5) * _R_MAX / (_Hb / 2 - 0.5)
    x = (xx - _Wb / 2 + 0.5) * _R_MAX / (_Wb / 2 - 0.5)
    phi = (np.arctan2(y, x) + 2 * np.pi) % (2 * np.pi)
    col = _Wr - 1 - phi / (2 * np.pi) * (_Wr - 1)
    ix = col * (_Wr - 1) / _Wr
    ix0 = np.floor(ix)
    w1 = (ix - ix0).astype(np.float32).reshape(-1)
    c0 = np.clip(ix0, 0, _Wr - 2).astype(np.int32).reshape(-1)
    return c0, w1


_C0_HOST, _W1_HOST = _precompute_grid()


@functools.cache
def _build_sc_kernel():
    mesh = plsc.VectorSubcoreMesh(core_axis_name="c", subcore_axis_name="s",
                                  num_cores=_NC, num_subcores=_NS)

    @functools.partial(
        pl.kernel,
        out_type=jax.ShapeDtypeStruct((_ROWS, _N), jnp.float32),
        mesh=mesh,
        compiler_params=pltpu.CompilerParams(needs_layout_passes=False),
        scratch_types=[
            pltpu.VMEM((_RPW * _Wr,), jnp.float32),
            pltpu.VMEM((2, _Wr), jnp.float32),
            pltpu.VMEM((2, _CH), jnp.int32),
            pltpu.VMEM((2, _CH), jnp.float32),
            pltpu.VMEM((2, _RPW, _CH), jnp.float32),
            pltpu.SemaphoreType.DMA,
            pltpu.SemaphoreType.DMA,
            pltpu.SemaphoreType.DMA,
            pltpu.SemaphoreType.DMA,
        ],
    )
    def _rv2bev_sc(rv_hbm, c0_hbm, w_hbm, out_hbm,
                   tab_v, pair_v, idx_v, w_v, out_v, isem_c, isem_w, osem0, osem1):
        _sc_body(rv_hbm, c0_hbm, w_hbm, out_hbm,
                 tab_v, pair_v, idx_v, w_v, out_v, isem_c, isem_w, osem0, osem1)

    return _rv2bev_sc


def _sc_body(rv_hbm, c0_hbm, w_hbm, out_hbm,
             tab_v, pair_v, idx_v, w_v, out_v, isem_c, isem_w, osem0, osem1):
    wid = lax.axis_index("s") * _NC + lax.axis_index("c")
    rg = wid % _RG
    pg = wid // _RG
    row0 = rg * _RPW
    px0 = pg * _PX_PER_W

    pltpu.async_copy(c0_hbm.at[pl.ds(px0, _CH)], idx_v.at[0], isem_c)
    pltpu.async_copy(w_hbm.at[pl.ds(px0, _CH)], w_v.at[0], isem_w)

    def load_row(r, carry):
        pltpu.sync_copy(rv_hbm.at[row0 + r], pair_v)

        def avg_vec(i, c):
            sl = pl.ds(i * _L, _L)
            tab_v[pl.ds(r * _Wr + i * _L, _L)] = (
                0.5 * (pair_v[0, sl] + pair_v[1, sl]))
            return c

        lax.fori_loop(0, _Wr // _L, avg_vec, 0)
        return carry

    lax.fori_loop(0, _RPW, load_row, 0)

    def out_block_wait(parity):
        dst = out_hbm.at[pl.ds(row0, _RPW), pl.ds(px0, _CH)]

        @pl.when(parity == 0)
        def _():
            pltpu.make_async_copy(out_v.at[0], dst, osem0).wait()

        @pl.when(parity == 1)
        def _():
            pltpu.make_async_copy(out_v.at[1], dst, osem1).wait()

    def chunk_body(j, carry):
        p = j % 2
        off = px0 + j * _CH
        pltpu.make_async_copy(c0_hbm.at[pl.ds(px0, _CH)], idx_v.at[p], isem_c).wait()
        pltpu.make_async_copy(w_hbm.at[pl.ds(px0, _CH)], w_v.at[p], isem_w).wait()

        @pl.when(j < _NCH - 1)
        def _():
            noff = off + _CH
            pltpu.async_copy(c0_hbm.at[pl.ds(noff, _CH)], idx_v.at[1 - p], isem_c)
            pltpu.async_copy(w_hbm.at[pl.ds(noff, _CH)], w_v.at[1 - p], isem_w)

        @pl.when(j >= 2)
        def _():
            out_block_wait(p)

        @plsc.parallel_loop(0, _CH // _L, unroll=2)
        def _gather(i):
            sl = pl.ds(i * _L, _L)
            idx = idx_v[p, sl]
            w1 = w_v[p, sl]
            w0 = 1.0 - w1
            for r in range(_RPW):
                ir_ = idx + (r * _Wr)
                g0 = plsc.load_gather(tab_v, [ir_])
                g1 = plsc.load_gather(tab_v, [ir_ + 1])
                out_v[p, r, sl] = w0 * g0 + w1 * g1

        dst = out_hbm.at[pl.ds(row0, _RPW), pl.ds(off, _CH)]

        @pl.when(p == 0)
        def _():
            pltpu.async_copy(out_v.at[0], dst, osem0)

        @pl.when(p == 1)
        def _():
            pltpu.async_copy(out_v.at[1], dst, osem1)

        return carry

    lax.fori_loop(0, _NCH, chunk_body, 0)

    out_block_wait(0)
    out_block_wait(1)


def kernel(rv_feat, ref_bev):
    del ref_bev
    rv_rows = rv_feat[:, :, _VERT_ROW - 1:_VERT_ROW + 1, :].reshape(_ROWS, 2, _Wr)
    c0 = jnp.asarray(_C0_HOST)
    w1 = jnp.asarray(_W1_HOST)
    out = _build_sc_kernel()(rv_rows, c0, w1)
    return out.reshape(_B, _C, _Hb, _Wb)


# speedup vs baseline: 30.0645x; 1.4767x over previous
import functools

import numpy as np
import jax
import jax.numpy as jnp
from jax import lax
from jax.experimental import pallas as pl
from jax.experimental.pallas import tpu as pltpu
from jax.experimental.pallas import tpu_sc as plsc

_Hr, _Wr = 64, 2048
_Hb, _Wb = 512, 512
_R_MAX = 50.0
_VERT_ROW = 32
_B, _C = 2, 64
_N = _Hb * _Wb

_NC, _NS, _L = 2, 16, 16
_NW = _NC * _NS
_ROWS = _B * _C
_RG = 16
_PG = _NW // _RG
_RPW = _ROWS // _RG
_PX_PER_W = _N // _PG
_CH = 4096
_NCH = _PX_PER_W // _CH
_YB = _CH // _Wb


def _precompute_grid():
    yy, xx = np.meshgrid(np.arange(_Hb, dtype=np.float64),
                         np.arange(_Wb, dtype=np.float64), indexing="ij")
    y = (yy - _Hb / 2 + 0.5) * _R_MAX / (_Hb / 2 - 0.5)
    x = (xx - _Wb / 2 + 0.5) * _R_MAX / (_Wb / 2 - 0.5)
    phi = (np.arctan2(y, x) + 2 * np.pi) % (2 * np.pi)
    col = _Wr - 1 - phi / (2 * np.pi) * (_Wr - 1)
    ix = col * (_Wr - 1) / _Wr
    ix0 = np.floor(ix)
    w1 = (ix - ix0).astype(np.float32).reshape(-1)
    c0 = np.clip(ix0, 0, _Wr - 2).astype(np.int32).reshape(-1)
    return c0, w1


_C0_HOST, _W1_HOST = _precompute_grid()


@functools.cache
def _build_sc_kernel():
    mesh = plsc.VectorSubcoreMesh(core_axis_name="c", subcore_axis_name="s",
                                  num_cores=_NC, num_subcores=_NS)

    @functools.partial(
        pl.kernel,
        out_type=jax.ShapeDtypeStruct((_B, _C, _Hb, _Wb), jnp.float32),
        mesh=mesh,
        compiler_params=pltpu.CompilerParams(needs_layout_passes=False),
        scratch_types=[
            pltpu.VMEM((_RPW * _Wr,), jnp.float32),
            pltpu.VMEM((2, _Wr), jnp.float32),
            pltpu.VMEM((2, _CH), jnp.int32),
            pltpu.VMEM((2, _CH), jnp.float32),
            pltpu.VMEM((2, _RPW, _YB, _Wb), jnp.float32),
            pltpu.SemaphoreType.DMA,
            pltpu.SemaphoreType.DMA,
            pltpu.SemaphoreType.DMA,
            pltpu.SemaphoreType.DMA,
        ],
    )
    def _rv2bev_sc(rv_hbm, c0_hbm, w_hbm, out_hbm,
                   tab_v, pair_v, idx_v, w_v, out_v, isem_c, isem_w, osem0, osem1):
        _sc_body(rv_hbm, c0_hbm, w_hbm, out_hbm,
                 tab_v, pair_v, idx_v, w_v, out_v, isem_c, isem_w, osem0, osem1)

    return _rv2bev_sc


def _sc_body(rv_hbm, c0_hbm, w_hbm, out_hbm,
             tab_v, pair_v, idx_v, w_v, out_v, isem_c, isem_w, osem0, osem1):
    wid = lax.axis_index("s") * _NC + lax.axis_index("c")
    rg = wid % _RG
    pg = wid // _RG
    row0 = rg * _RPW
    bi = rg // (_RG // _B)
    ci = (rg % (_RG // _B)) * _RPW
    px0 = pg * _PX_PER_W
    y0w = px0 // _Wb

    pltpu.async_copy(c0_hbm.at[pl.ds(px0, _CH)], idx_v.at[0], isem_c)
    pltpu.async_copy(w_hbm.at[pl.ds(px0, _CH)], w_v.at[0], isem_w)

    def load_row(r, carry):
        pltpu.sync_copy(rv_hbm.at[row0 + r], pair_v)

        def avg_vec(i, c):
            sl = pl.ds(i * _L, _L)
            tab_v[pl.ds(r * _Wr + i * _L, _L)] = (
                0.5 * (pair_v[0, sl] + pair_v[1, sl]))
            return c

        lax.fori_loop(0, _Wr // _L, avg_vec, 0)
        return carry

    lax.fori_loop(0, _RPW, load_row, 0)

    def out_block_wait(parity):
        dst = out_hbm.at[bi, pl.ds(ci, _RPW), pl.ds(pl.multiple_of(y0w, _YB), _YB), :]

        @pl.when(parity == 0)
        def _():
            pltpu.make_async_copy(out_v.at[0], dst, osem0).wait()

        @pl.when(parity == 1)
        def _():
            pltpu.make_async_copy(out_v.at[1], dst, osem1).wait()

    def chunk_body(j, carry):
        p = j % 2
        off = px0 + j * _CH
        pltpu.make_async_copy(c0_hbm.at[pl.ds(px0, _CH)], idx_v.at[p], isem_c).wait()
        pltpu.make_async_copy(w_hbm.at[pl.ds(px0, _CH)], w_v.at[p], isem_w).wait()

        @pl.when(j < _NCH - 1)
        def _():
            noff = off + _CH
            pltpu.async_copy(c0_hbm.at[pl.ds(noff, _CH)], idx_v.at[1 - p], isem_c)
            pltpu.async_copy(w_hbm.at[pl.ds(noff, _CH)], w_v.at[1 - p], isem_w)

        @pl.when(j >= 2)
        def _():
            out_block_wait(p)

        @plsc.parallel_loop(0, _CH // _L, unroll=2)
        def _gather(i):
            ys = lax.shift_right_logical(i, 5)
            x0 = lax.shift_left(lax.bitwise_and(i, 31), 4)
            sl = pl.ds(i * _L, _L)
            xsl = pl.ds(x0, _L)
            idx = idx_v[p, sl]
            w1 = w_v[p, sl]
            w0 = 1.0 - w1
            for r in range(_RPW):
                ir_ = idx + (r * _Wr)
                g0 = plsc.load_gather(tab_v, [ir_])
                g1 = plsc.load_gather(tab_v, [ir_ + 1])
                out_v[p, r, ys, xsl] = w0 * g0 + w1 * g1

        yc = pl.multiple_of(y0w + j * _YB, _YB)
        dst = out_hbm.at[bi, pl.ds(ci, _RPW), pl.ds(yc, _YB), :]

        @pl.when(p == 0)
        def _():
            pltpu.async_copy(out_v.at[0], dst, osem0)

        @pl.when(p == 1)
        def _():
            pltpu.async_copy(out_v.at[1], dst, osem1)

        return carry

    lax.fori_loop(0, _NCH, chunk_body, 0)

    out_block_wait(0)
    out_block_wait(1)


def kernel(rv_feat, ref_bev):
    del ref_bev
    rv_rows = rv_feat[:, :, _VERT_ROW - 1:_VERT_ROW + 1, :].reshape(_ROWS, 2, _Wr)
    c0 = jnp.asarray(_C0_HOST)
    w1 = jnp.asarray(_W1_HOST)
    return _build_sc_kernel()(rv_rows, c0, w1)


# speedup vs baseline: 31.2284x; 1.0387x over previous
import functools

import numpy as np
import jax
import jax.numpy as jnp
from jax import lax
from jax.experimental import pallas as pl
from jax.experimental.pallas import tpu as pltpu
from jax.experimental.pallas import tpu_sc as plsc

_Hr, _Wr = 64, 2048
_Hb, _Wb = 512, 512
_R_MAX = 50.0
_VERT_ROW = 32
_B, _C = 2, 64
_N = _Hb * _Wb

_NC, _NS, _L = 2, 16, 16
_NW = _NC * _NS
_ROWS = _B * _C
_RG = 16
_PG = _NW // _RG
_RPW = _ROWS // _RG
_PX_PER_W = _N // _PG
_CH = 4096
_NCH = _PX_PER_W // _CH
_YB = _CH // _Wb


def _precompute_grid():
    yy, xx = np.meshgrid(np.arange(_Hb, dtype=np.float64),
                         np.arange(_Wb, dtype=np.float64), indexing="ij")
    y = (yy - _Hb / 2 + 0.5) * _R_MAX / (_Hb / 2 - 0.5)
    x = (xx - _Wb / 2 + 0.5) * _R_MAX / (_Wb / 2 - 0.5)
    phi = (np.arctan2(y, x) + 2 * np.pi) % (2 * np.pi)
    col = _Wr - 1 - phi / (2 * np.pi) * (_Wr - 1)
    ix = col * (_Wr - 1) / _Wr
    ix0 = np.floor(ix)
    w1 = (ix - ix0).astype(np.float32).reshape(-1)
    c0 = np.clip(ix0, 0, _Wr - 2).astype(np.int32).reshape(-1)
    return c0, w1


_C0_HOST, _W1_HOST = _precompute_grid()


@functools.cache
def _build_sc_kernel():
    mesh = plsc.VectorSubcoreMesh(core_axis_name="c", subcore_axis_name="s",
                                  num_cores=_NC, num_subcores=_NS)

    @functools.partial(
        pl.kernel,
        out_type=jax.ShapeDtypeStruct((_B, _C, _Hb, _Wb), jnp.float32),
        mesh=mesh,
        compiler_params=pltpu.CompilerParams(needs_layout_passes=False),
        scratch_types=[
            pltpu.VMEM((_RPW * _Wr,), jnp.float32),
            pltpu.VMEM((2, _Wr), jnp.float32),
            pltpu.VMEM((2, _CH), jnp.int32),
            pltpu.VMEM((2, _CH), jnp.float32),
            pltpu.VMEM((2, _RPW, _YB, _Wb), jnp.float32),
            pltpu.SemaphoreType.DMA,
            pltpu.SemaphoreType.DMA,
            pltpu.SemaphoreType.DMA,
            pltpu.SemaphoreType.DMA,
        ],
    )
    def _rv2bev_sc(rv_hbm, c0_hbm, w_hbm, out_hbm,
                   tab_v, pair_v, idx_v, w_v, out_v, isem_c, isem_w, osem0, osem1):
        _sc_body(rv_hbm, c0_hbm, w_hbm, out_hbm,
                 tab_v, pair_v, idx_v, w_v, out_v, isem_c, isem_w, osem0, osem1)

    return _rv2bev_sc


def _sc_body(rv_hbm, c0_hbm, w_hbm, out_hbm,
             tab_v, pair_v, idx_v, w_v, out_v, isem_c, isem_w, osem0, osem1):
    wid = lax.axis_index("s") * _NC + lax.axis_index("c")
    rg = wid % _RG
    pg = wid // _RG
    row0 = rg * _RPW
    bi = rg // (_RG // _B)
    ci = (rg % (_RG // _B)) * _RPW
    px0 = pg * _PX_PER_W
    y0w = px0 // _Wb

    pltpu.async_copy(c0_hbm.at[pl.ds(px0, _CH)], idx_v.at[0], isem_c)
    pltpu.async_copy(w_hbm.at[pl.ds(px0, _CH)], w_v.at[0], isem_w)

    def load_row(r, carry):
        pltpu.sync_copy(rv_hbm.at[row0 + r], pair_v)

        def avg_vec(i, c):
            sl = pl.ds(i * _L, _L)
            tab_v[pl.ds(r * _Wr + i * _L, _L)] = (
                0.5 * (pair_v[0, sl] + pair_v[1, sl]))
            return c

        lax.fori_loop(0, _Wr // _L, avg_vec, 0)
        return carry

    lax.fori_loop(0, _RPW, load_row, 0)

    def out_block_wait(parity):
        dst = out_hbm.at[bi, pl.ds(ci, _RPW), pl.ds(pl.multiple_of(y0w, _YB), _YB), :]

        @pl.when(parity == 0)
        def _():
            pltpu.make_async_copy(out_v.at[0], dst, osem0).wait()

        @pl.when(parity == 1)
        def _():
            pltpu.make_async_copy(out_v.at[1], dst, osem1).wait()

    def chunk_pair(j2, carry):
        for p in (0, 1):
            j = 2 * j2 + p
            off = px0 + j * _CH
            pltpu.make_async_copy(c0_hbm.at[pl.ds(px0, _CH)], idx_v.at[p], isem_c).wait()
            pltpu.make_async_copy(w_hbm.at[pl.ds(px0, _CH)], w_v.at[p], isem_w).wait()

            @pl.when(j < _NCH - 1)
            def _(off=off, p=p):
                noff = off + _CH
                pltpu.async_copy(c0_hbm.at[pl.ds(noff, _CH)], idx_v.at[1 - p], isem_c)
                pltpu.async_copy(w_hbm.at[pl.ds(noff, _CH)], w_v.at[1 - p], isem_w)

            @pl.when(j2 >= 1)
            def _(p=p):
                out_block_wait(p)

            for ys in range(_YB):
                @plsc.parallel_loop(0, _Wb // _L, unroll=2)
                def _gather(iv, p=p, ys=ys):
                    sl = pl.ds(ys * _Wb + iv * _L, _L)
                    idx = idx_v[p, sl]
                    idx1 = idx + 1
                    w1 = w_v[p, sl]
                    w0 = 1.0 - w1
                    for r in range(_RPW):
                        row = tab_v.at[pl.ds(r * _Wr, _Wr)]
                        g0 = plsc.load_gather(row, [idx])
                        g1 = plsc.load_gather(row, [idx1])
                        out_v[p, r, ys, pl.ds(iv * _L, _L)] = w0 * g0 + w1 * g1

            yc = pl.multiple_of(y0w + j * _YB, _YB)
            dst = out_hbm.at[bi, pl.ds(ci, _RPW), pl.ds(yc, _YB), :]
            if p == 0:
                pltpu.async_copy(out_v.at[0], dst, osem0)
            else:
                pltpu.async_copy(out_v.at[1], dst, osem1)

        return carry

    lax.fori_loop(0, _NCH // 2, chunk_pair, 0)

    out_block_wait(0)
    out_block_wait(1)


def kernel(rv_feat, ref_bev):
    del ref_bev
    rv_rows = rv_feat[:, :, _VERT_ROW - 1:_VERT_ROW + 1, :].reshape(_ROWS, 2, _Wr)
    c0 = jnp.asarray(_C0_HOST)
    w1 = jnp.asarray(_W1_HOST)
    return _build_sc_kernel()(rv_rows, c0, w1)
